# baseline (speedup 1.0000x reference)
import functools

import jax
import jax.numpy as jnp
from jax import lax
from jax.experimental import pallas as pl
from jax.experimental.pallas import tpu as pltpu
from jax.experimental.pallas import tpu_sc as plsc

N_NODES = 10000
N_EDGES = 320000
DIM = 128

NC = 2
NS = 16
NW = NC * NS

BATCH = 128
CH = 32
NB_CORE = (160, 0)
TOT_B = NS * (NB_CORE[0] + NB_CORE[1])
E_PAD = TOT_B * BATCH
OUT_PAD = 10240
STRIPE = OUT_PAD // NS


def _linear(h, W, b):
    def mm(h_ref, w_ref, b_ref, o_ref):
        acc = lax.dot_general(h_ref[...], w_ref[...],
                              (((1,), (1,)), ((), ())),
                              preferred_element_type=jnp.float32)
        acc = acc + b_ref[0][None, :]
        o_ref[0] = acc
        o_ref[1] = acc

    b8 = jnp.broadcast_to(b[None, :], (8, DIM))
    return pl.pallas_call(
        mm,
        grid=(10,),
        in_specs=[
            pl.BlockSpec((1000, DIM), lambda i: (i, 0)),
            pl.BlockSpec((DIM, DIM), lambda i: (0, 0)),
            pl.BlockSpec((8, DIM), lambda i: (0, 0)),
        ],
        out_specs=pl.BlockSpec((NC, 1000, DIM), lambda i: (0, i, 0)),
        out_shape=jax.ShapeDtypeStruct((NC, N_NODES, DIM), jnp.float32),
    )(h, W, b8)


def _make_aggregate():
    mesh = plsc.VectorSubcoreMesh(core_axis_name="c", subcore_axis_name="s")

    @functools.partial(
        pl.kernel,
        mesh=mesh,
        out_type=jax.ShapeDtypeStruct((NC, OUT_PAD, DIM), jnp.float32),
        scratch_types=[
            pltpu.VMEM((CH, BATCH), jnp.int32),
            pltpu.VMEM((CH, BATCH), jnp.int32),
            pltpu.VMEM((BATCH, DIM), jnp.float32),
            pltpu.VMEM((BATCH, DIM), jnp.float32),
            pltpu.VMEM_SHARED((OUT_PAD, DIM), jnp.float32),
            pltpu.SemaphoreType.DMA,
            pltpu.SemaphoreType.DMA,
        ],
    )
    def agg(h2_hbm, src_hbm, dst_hbm, out_hbm,
            src_v, dst_v, rows_a, rows_b, acc_sh, g_a, g_b):
        c = lax.axis_index("c")
        s = lax.axis_index("s")

        z16 = jnp.zeros((16,), jnp.float32)

        def zrow(i, _):
            for cc in range(DIM // 16):
                rows_a[i, pl.ds(cc * 16, 16)] = z16
            return _

        with jax.named_scope("zinit"):
            lax.fori_loop(0, BATCH, zrow, None)
            row0 = s * STRIPE
            for k in range(STRIPE // BATCH):
                pltpu.sync_copy(rows_a,
                                acc_sh.at[pl.ds(row0 + k * BATCH, BATCH)])
            plsc.subcore_barrier()

        nch = jnp.where(c == 0, NB_CORE[0] // CH, NB_CORE[1] // CH)
        base_b = jnp.where(c == 0, s * NB_CORE[0],
                           NS * NB_CORE[0] + s * NB_CORE[1])

        def chunk_body(kk, _):
            off = base_b + kk * CH
            pltpu.sync_copy(src_hbm.at[pl.ds(off, CH)], src_v)
            pltpu.sync_copy(dst_hbm.at[pl.ds(off, CH)], dst_v)
            pltpu.async_copy(h2_hbm.at[src_v.at[0]], rows_a, g_a)

            def body(i, __):
                b0 = 2 * i
                bp = jnp.minimum(b0 + 2, CH - 1)
                pltpu.make_async_copy(h2_hbm.at[src_v.at[b0]], rows_a, g_a
                                      ).wait()
                pltpu.async_copy(h2_hbm.at[src_v.at[b0 + 1]], rows_b, g_b)
                pltpu.sync_copy(rows_a, acc_sh.at[dst_v.at[b0]], add=True)
                pltpu.make_async_copy(h2_hbm.at[src_v.at[b0 + 1]], rows_b, g_b
                                      ).wait()
                pltpu.async_copy(h2_hbm.at[src_v.at[bp]], rows_a, g_a)
                pltpu.sync_copy(rows_b, acc_sh.at[dst_v.at[b0 + 1]], add=True)
                return __

            lax.fori_loop(0, CH // 2, body, None)
            pltpu.make_async_copy(h2_hbm.at[src_v.at[CH - 1]], rows_a, g_a
                                  ).wait()
            return _

        with jax.named_scope("edges"):
            lax.fori_loop(0, nch, chunk_body, None)
        with jax.named_scope("wb"):
            plsc.subcore_barrier()
            pltpu.sync_copy(acc_sh.at[pl.ds(row0, STRIPE)],
                            out_hbm.at[c, pl.ds(row0, STRIPE)])

    return agg


_aggregate_sc = _make_aggregate()


def _combine(partials):
    def add2(p_ref, o_ref):
        o_ref[...] = p_ref[0] + p_ref[1]

    return pl.pallas_call(
        add2,
        grid=(10,),
        in_specs=[pl.BlockSpec((NC, 1000, DIM), lambda i: (0, i, 0))],
        out_specs=pl.BlockSpec((1000, DIM), lambda i: (i, 0)),
        out_shape=jax.ShapeDtypeStruct((N_NODES, DIM), jnp.float32),
    )(partials)


def kernel(h, edge_index, W, b):
    h2 = _linear(h, W, b)

    dst = edge_index[0].astype(jnp.int32)
    src = edge_index[1].astype(jnp.int32)
    pad = E_PAD - N_EDGES
    src_p = jnp.concatenate([src, jnp.zeros((pad,), jnp.int32)])
    dst_p = jnp.concatenate([dst, jnp.full((pad,), N_NODES, jnp.int32)])
    src_p = src_p.reshape(TOT_B, BATCH)
    dst_p = dst_p.reshape(TOT_B, BATCH)
    core1 = (jnp.arange(TOT_B, dtype=jnp.int32) >= NS * NB_CORE[0])
    src_p = src_p + jnp.where(core1, N_NODES, 0).astype(jnp.int32)[:, None]

    partials = _aggregate_sc(h2.reshape(NC * N_NODES, DIM), src_p, dst_p)
    return _combine(partials)


# speedup vs baseline: 1.4131x; 1.4131x over previous
import functools

import jax
import jax.numpy as jnp
from jax import lax
from jax.experimental import pallas as pl
from jax.experimental.pallas import tpu as pltpu
from jax.experimental.pallas import tpu_sc as plsc

N_NODES = 10000
N_EDGES = 320000
DIM = 128

NC = 2
NS = 16
NW = NC * NS

BATCH = 128
CH = 32
NB_CORE = (128, 32)
TOT_B = NS * (NB_CORE[0] + NB_CORE[1])
E_PAD = TOT_B * BATCH
OUT_PAD = 10240
STRIPE = OUT_PAD // NS


def _linear(h, W, b):
    def mm(h_ref, w_ref, b_ref, o_ref):
        acc = lax.dot_general(h_ref[...], w_ref[...],
                              (((1,), (1,)), ((), ())),
                              preferred_element_type=jnp.float32)
        acc = acc + b_ref[0][None, :]
        o_ref[0] = acc
        o_ref[1] = acc

    b8 = jnp.broadcast_to(b[None, :], (8, DIM))
    return pl.pallas_call(
        mm,
        grid=(10,),
        in_specs=[
            pl.BlockSpec((1000, DIM), lambda i: (i, 0)),
            pl.BlockSpec((DIM, DIM), lambda i: (0, 0)),
            pl.BlockSpec((8, DIM), lambda i: (0, 0)),
        ],
        out_specs=pl.BlockSpec((NC, 1000, DIM), lambda i: (0, i, 0)),
        out_shape=jax.ShapeDtypeStruct((NC, N_NODES, DIM), jnp.float32),
    )(h, W, b8)


def _make_aggregate():
    mesh = plsc.VectorSubcoreMesh(core_axis_name="c", subcore_axis_name="s")

    @functools.partial(
        pl.kernel,
        mesh=mesh,
        out_type=jax.ShapeDtypeStruct((NC, OUT_PAD, DIM), jnp.float32),
        scratch_types=[
            pltpu.VMEM((CH, BATCH), jnp.int32),
            pltpu.VMEM((CH, BATCH), jnp.int32),
            pltpu.VMEM((BATCH, DIM), jnp.float32),
            pltpu.VMEM((BATCH, DIM), jnp.float32),
            pltpu.VMEM_SHARED((OUT_PAD, DIM), jnp.float32),
            pltpu.SemaphoreType.DMA,
            pltpu.SemaphoreType.DMA,
            pltpu.SemaphoreType.DMA,
            pltpu.SemaphoreType.DMA,
        ],
    )
    def agg(h2_hbm, src_hbm, dst_hbm, out_hbm,
            src_v, dst_v, rows_a, rows_b, acc_sh, g_a0, g_a1, g_b0, g_b1):
        c = lax.axis_index("c")
        s = lax.axis_index("s")

        z16 = jnp.zeros((16,), jnp.float32)

        def zrow(i, _):
            for cc in range(DIM // 16):
                rows_a[i, pl.ds(cc * 16, 16)] = z16
            return _

        with jax.named_scope("zinit"):
            lax.fori_loop(0, BATCH, zrow, None)
            row0 = s * STRIPE
            for k in range(STRIPE // BATCH):
                pltpu.sync_copy(rows_a,
                                acc_sh.at[pl.ds(row0 + k * BATCH, BATCH)])
            plsc.subcore_barrier()

        nch = jnp.where(c == 0, NB_CORE[0] // CH, NB_CORE[1] // CH)
        base_b = jnp.where(c == 0, s * NB_CORE[0],
                           NS * NB_CORE[0] + s * NB_CORE[1])

        def chunk_body(kk, _):
            off = base_b + kk * CH
            pltpu.sync_copy(src_hbm.at[pl.ds(off, CH)], src_v)
            pltpu.sync_copy(dst_hbm.at[pl.ds(off, CH)], dst_v)
            def gat(buf, sem0, sem1, b):
                pltpu.async_copy(h2_hbm.at[src_v.at[b, pl.ds(0, 64)]],
                                 buf.at[pl.ds(0, 64)], sem0)
                pltpu.async_copy(h2_hbm.at[src_v.at[b, pl.ds(64, 64)]],
                                 buf.at[pl.ds(64, 64)], sem1)

            def gwait(buf, sem0, sem1, b):
                pltpu.make_async_copy(h2_hbm.at[src_v.at[b, pl.ds(0, 64)]],
                                      buf.at[pl.ds(0, 64)], sem0).wait()
                pltpu.make_async_copy(h2_hbm.at[src_v.at[b, pl.ds(64, 64)]],
                                      buf.at[pl.ds(64, 64)], sem1).wait()

            gat(rows_a, g_a0, g_a1, 0)

            def body(i, __):
                b0 = 2 * i
                bp = jnp.minimum(b0 + 2, CH - 1)
                gwait(rows_a, g_a0, g_a1, b0)
                gat(rows_b, g_b0, g_b1, b0 + 1)
                pltpu.sync_copy(rows_a, acc_sh.at[dst_v.at[b0]], add=True)
                gwait(rows_b, g_b0, g_b1, b0 + 1)
                gat(rows_a, g_a0, g_a1, bp)
                pltpu.sync_copy(rows_b, acc_sh.at[dst_v.at[b0 + 1]], add=True)
                return __

            lax.fori_loop(0, CH // 2, body, None)
            gwait(rows_a, g_a0, g_a1, CH - 1)
            return _

        with jax.named_scope("edges"):
            lax.fori_loop(0, nch, chunk_body, None)
        with jax.named_scope("wb"):
            plsc.subcore_barrier()
            pltpu.sync_copy(acc_sh.at[pl.ds(row0, STRIPE)],
                            out_hbm.at[c, pl.ds(row0, STRIPE)])

    return agg


_aggregate_sc = _make_aggregate()


def _combine(partials):
    def add2(p_ref, o_ref):
        o_ref[...] = p_ref[0] + p_ref[1]

    return pl.pallas_call(
        add2,
        grid=(10,),
        in_specs=[pl.BlockSpec((NC, 1000, DIM), lambda i: (0, i, 0))],
        out_specs=pl.BlockSpec((1000, DIM), lambda i: (i, 0)),
        out_shape=jax.ShapeDtypeStruct((N_NODES, DIM), jnp.float32),
    )(partials)


def kernel(h, edge_index, W, b):
    h2 = _linear(h, W, b)

    dst = edge_index[0].astype(jnp.int32)
    src = edge_index[1].astype(jnp.int32)
    pad = E_PAD - N_EDGES
    src_p = jnp.concatenate([src, jnp.zeros((pad,), jnp.int32)])
    dst_p = jnp.concatenate([dst, jnp.full((pad,), N_NODES, jnp.int32)])
    src_p = src_p.reshape(TOT_B, BATCH)
    dst_p = dst_p.reshape(TOT_B, BATCH)
    core1 = (jnp.arange(TOT_B, dtype=jnp.int32) >= NS * NB_CORE[0])
    src_p = src_p + jnp.where(core1, N_NODES, 0).astype(jnp.int32)[:, None]

    partials = _aggregate_sc(h2.reshape(NC * N_NODES, DIM), src_p, dst_p)
    return _combine(partials)
